# Initial kernel scaffold; baseline (speedup 1.0000x reference)
import jax
import jax.numpy as jnp
from jax.experimental import pallas as pl


def kernel(src_node_values, src_coords, src_batch, tgt_node_values, tgt_coords, tgt_batch, W_enc, b_enc, W_rel, b_rel, W_root, W_skip):
    raise NotImplementedError("write your pallas kernel here")


# speedup vs baseline: 392.2828x; 392.2828x over previous
import functools

import jax
import jax.numpy as jnp
from jax import lax
from jax.experimental import pallas as pl
from jax.experimental.pallas import tpu as pltpu
from jax.experimental.pallas import tpu_sc as plsc

N = 50000
NTOT = 2 * N
NS = 102400
PAD = NS - NTOT
C = 128
NGRAPH = 16
NXY = 32
BIG = 1 << 30
RT = 128
NT = NS // RT
NW = 32
PERW = NS // NW
KROW = PERW // 128


def _cluster_body(xs_ref, ys_ref, b_ref, sub_ref):
    xs = xs_ref[...]
    ys = ys_ref[...]
    b = b_ref[...]
    big = jnp.float32(3.0e38)
    lox = jnp.zeros_like(xs)
    loy = jnp.zeros_like(ys)
    hix = jnp.zeros_like(xs)
    hiy = jnp.zeros_like(ys)
    for g in range(2 * NGRAPH):
        m = b == g
        minx = jnp.min(jnp.where(m, xs, big))
        maxx = jnp.max(jnp.where(m, xs, -big))
        miny = jnp.min(jnp.where(m, ys, big))
        maxy = jnp.max(jnp.where(m, ys, -big))
        lox = jnp.where(m, minx, lox)
        hix = jnp.where(m, maxx, hix)
        loy = jnp.where(m, miny, loy)
        hiy = jnp.where(m, maxy, hiy)
    nx = (xs - lox) / jnp.maximum(hix - lox, 1e-12)
    ny = (ys - loy) / jnp.maximum(hiy - loy, 1e-12)
    cx = jnp.clip(jnp.floor(nx * NXY).astype(jnp.int32), 0, NXY - 1)
    cy = jnp.clip(jnp.floor(ny * NXY).astype(jnp.int32), 0, NXY - 1)
    sub = (b % NGRAPH) * (NXY * NXY) + cy * NXY + cx
    sub_ref[...] = jnp.where(b < 2 * NGRAPH, sub, BIG)


def _enc_body(nv_ref, pos_ref, w1_ref, w2_ref, be_ref, h_ref):
    nv = nv_ref[...]
    px = pos_ref[:, 0:1]
    py = pos_ref[:, 1:2]
    h = jnp.dot(nv, w1_ref[...], preferred_element_type=jnp.float32)
    h = h + px * w2_ref[0:1, :] + py * w2_ref[1:2, :] + be_ref[...]
    h_ref[...] = h


def _band_body(hp_ref, hc_ref, hn_ref, axc_ref, tp_ref, tc_ref, tn_ref,
               wrel_ref, wroot_ref, brel_ref, perm_ref, out_ref, sidx_ref):
    r = pl.program_id(0)
    sub_r = axc_ref[:, 2:3]
    px_r = axc_ref[:, 0:1]
    py_r = axc_ref[:, 1:2]
    rpos = r * RT + lax.broadcasted_iota(jnp.int32, (RT, 1), 0)
    acc = jnp.zeros((RT, C), jnp.float32)
    cnt = jnp.zeros((RT, 1), jnp.float32)
    blocks = (
        (hp_ref, tp_ref, jnp.maximum(r - 1, 0), r > 0),
        (hc_ref, tc_ref, r, True),
        (hn_ref, tn_ref, jnp.minimum(r + 1, NT - 1), r < NT - 1),
    )
    for h_ref, t_ref, cb, valid in blocks:
        t = t_ref[...]
        cpos = cb * RT + lax.broadcasted_iota(jnp.int32, (1, RT), 1)
        m = (sub_r == t[2:3, :]) & (rpos != cpos) & valid
        dx = px_r - t[0:1, :]
        dy = py_r - t[1:2, :]
        w = jnp.sqrt(dx * dx + dy * dy + 1e-12)
        wm = jnp.where(m, w, 0.0)
        acc = acc + jnp.dot(wm, h_ref[...], preferred_element_type=jnp.float32)
        cnt = cnt + jnp.sum(m.astype(jnp.float32), axis=1, keepdims=True)
    aggr = acc / jnp.maximum(cnt, 1.0)
    out = jnp.dot(aggr, wrel_ref[...], preferred_element_type=jnp.float32)
    out = out + jnp.dot(hc_ref[...], wroot_ref[...],
                        preferred_element_type=jnp.float32) + brel_ref[...]
    out_ref[...] = out
    pp = perm_ref[...]
    ii = lax.broadcasted_iota(jnp.int32, (1, 1, RT), 2)
    dump = N + ((r * RT + ii) & 1023)
    is_tgt = (pp >= N) & (pp < NTOT)
    sidx_ref[...] = jnp.where(is_tgt, pp - N, dump)


def _final_body(tnv_ref, buf_ref, wskip_ref, out_ref):
    out_ref[...] = buf_ref[...] + jnp.dot(
        tnv_ref[...], wskip_ref[...], preferred_element_type=jnp.float32)


def _gather_body(h_hbm, aux_hbm, perm_hbm, hs_hbm, auxs_hbm,
                 idx_v, hrow_v, arow_v, sem1, sem2):
    nc = 2
    wid = lax.axis_index("s") * nc + lax.axis_index("c")
    pltpu.sync_copy(perm_hbm.at[wid], idx_v)

    def step(j, carry):
        row = idx_v.at[j]
        cp1 = pltpu.async_copy(h_hbm.at[row], hrow_v, sem1)
        cp2 = pltpu.async_copy(aux_hbm.at[row], arow_v, sem2)
        cp1.wait()
        cp2.wait()
        base = wid * PERW + j * 128
        pltpu.sync_copy(hrow_v, hs_hbm.at[pl.ds(base, 128)])
        pltpu.sync_copy(arow_v, auxs_hbm.at[pl.ds(base, 128)])
        return carry

    lax.fori_loop(0, KROW, step, 0)


def _scatter_body(outs_hbm, sidx_hbm, buf_hbm, idx_v, row_v, sem):
    nc = 2
    wid = lax.axis_index("s") * nc + lax.axis_index("c")
    pltpu.sync_copy(sidx_hbm.at[wid], idx_v)

    def step(j, carry):
        base = wid * PERW + j * 128
        pltpu.sync_copy(outs_hbm.at[pl.ds(base, 128)], row_v)
        pltpu.async_copy(row_v, buf_hbm.at[idx_v.at[j]], sem).wait()
        return carry

    lax.fori_loop(0, KROW, step, 0)


def kernel(src_node_values, src_coords, src_batch, tgt_node_values, tgt_coords,
           tgt_batch, W_enc, b_enc, W_rel, b_rel, W_root, W_skip):
    f32 = jnp.float32
    i32 = jnp.int32

    coords = jnp.concatenate(
        [src_coords, tgt_coords, jnp.zeros((PAD, 2), f32)], axis=0)
    batch32 = jnp.concatenate(
        [src_batch, tgt_batch + NGRAPH, jnp.full((PAD,), 2 * NGRAPH, i32)])
    nv = jnp.concatenate(
        [src_node_values, tgt_node_values, jnp.zeros((PAD, C), f32)], axis=0)

    sub2 = pl.pallas_call(
        _cluster_body,
        out_shape=jax.ShapeDtypeStruct((128, NS // 128), i32),
    )(coords[:, 0].reshape(128, NS // 128),
      coords[:, 1].reshape(128, NS // 128),
      batch32.reshape(128, NS // 128))
    subp = sub2.reshape(NS)

    perm = jnp.argsort(subp).astype(i32)

    h = pl.pallas_call(
        _enc_body,
        grid=(NS // 512,),
        in_specs=[
            pl.BlockSpec((512, C), lambda r: (r, 0)),
            pl.BlockSpec((512, 2), lambda r: (r, 0)),
            pl.BlockSpec((C, C), lambda r: (0, 0)),
            pl.BlockSpec((2, C), lambda r: (0, 0)),
            pl.BlockSpec((1, C), lambda r: (0, 0)),
        ],
        out_specs=pl.BlockSpec((512, C), lambda r: (r, 0)),
        out_shape=jax.ShapeDtypeStruct((NS, C), f32),
    )(nv, coords, W_enc[:C], W_enc[C:C + 2], b_enc.reshape(1, C))

    aux = jnp.concatenate(
        [coords, subp.astype(f32)[:, None], jnp.zeros((NS, 125), f32)], axis=1)
    mesh = plsc.VectorSubcoreMesh(core_axis_name="c", subcore_axis_name="s",
                                  num_cores=2, num_subcores=16)
    gather_k = pl.kernel(
        _gather_body,
        out_type=[jax.ShapeDtypeStruct((NS, C), f32),
                  jax.ShapeDtypeStruct((NS, 128), f32)],
        mesh=mesh,
        scratch_types=[pltpu.VMEM((KROW, 128), i32),
                       pltpu.VMEM((128, C), f32),
                       pltpu.VMEM((128, 128), f32),
                       pltpu.SemaphoreType.DMA,
                       pltpu.SemaphoreType.DMA],
    )
    hs, auxs = gather_k(h, aux, perm.reshape(NW, KROW, 128))

    auxT = auxs[:, :8].T
    outs, sidx3 = pl.pallas_call(
        _band_body,
        grid=(NT,),
        in_specs=[
            pl.BlockSpec((RT, C), lambda r: (jnp.maximum(r - 1, 0), 0)),
            pl.BlockSpec((RT, C), lambda r: (r, 0)),
            pl.BlockSpec((RT, C), lambda r: (jnp.minimum(r + 1, NT - 1), 0)),
            pl.BlockSpec((RT, 128), lambda r: (r, 0)),
            pl.BlockSpec((8, RT), lambda r: (0, jnp.maximum(r - 1, 0))),
            pl.BlockSpec((8, RT), lambda r: (0, r)),
            pl.BlockSpec((8, RT), lambda r: (0, jnp.minimum(r + 1, NT - 1))),
            pl.BlockSpec((C, C), lambda r: (0, 0)),
            pl.BlockSpec((C, C), lambda r: (0, 0)),
            pl.BlockSpec((1, C), lambda r: (0, 0)),
            pl.BlockSpec((1, 1, RT), lambda r: (r, 0, 0)),
        ],
        out_specs=[
            pl.BlockSpec((RT, C), lambda r: (r, 0)),
            pl.BlockSpec((1, 1, RT), lambda r: (r, 0, 0)),
        ],
        out_shape=[jax.ShapeDtypeStruct((NS, C), f32),
                   jax.ShapeDtypeStruct((NT, 1, RT), i32)],
    )(hs, hs, hs, auxs, auxT, auxT, auxT, W_rel, W_root,
      b_rel.reshape(1, C), perm.reshape(NT, 1, RT))

    scatter_k = pl.kernel(
        _scatter_body,
        out_type=jax.ShapeDtypeStruct((N + 1280, C), f32),
        mesh=mesh,
        scratch_types=[pltpu.VMEM((KROW, 128), i32),
                       pltpu.VMEM((128, C), f32),
                       pltpu.SemaphoreType.DMA],
    )
    buf = scatter_k(outs, sidx3.reshape(NW, KROW, 128))

    tgt_values = pl.pallas_call(
        _final_body,
        grid=(125,),
        in_specs=[
            pl.BlockSpec((400, C), lambda r: (r, 0)),
            pl.BlockSpec((400, C), lambda r: (r, 0)),
            pl.BlockSpec((C, C), lambda r: (0, 0)),
        ],
        out_specs=pl.BlockSpec((400, C), lambda r: (r, 0)),
        out_shape=jax.ShapeDtypeStruct((N, C), f32),
    )(tgt_node_values, buf, W_skip)
    return tgt_values
